# Initial kernel scaffold; baseline (speedup 1.0000x reference)
import jax
import jax.numpy as jnp
from jax.experimental import pallas as pl


def kernel(x, W_in, b_in, codebooks, W_out, b_out):
    raise NotImplementedError("write your pallas kernel here")


# speedup vs baseline: 1.2296x; 1.2296x over previous
import functools

import jax
import jax.numpy as jnp
from jax.experimental import pallas as pl

_B, _L, _D = 32, 1024, 768
_DV, _K, _NQ = 64, 512, 4
_N = _B * _L
_T = 1024


def _rvq_body(x_ref, win_ref, bin_ref, cb_ref, wout_ref, bout_ref,
              z_ref, idx_ref, loss_ref):
    i = pl.program_id(0)

    @pl.when(i == 0)
    def _init():
        loss_ref[...] = jnp.zeros_like(loss_ref)

    xb = x_ref[...]
    xp = jax.lax.dot_general(
        xb.astype(jnp.bfloat16), win_ref[...].astype(jnp.bfloat16),
        (((1,), (0,)), ((), ())),
        preferred_element_type=jnp.float32)
    res = xp + bin_ref[...]

    qsum = jnp.zeros_like(res)
    loss = jnp.float32(0.0)
    idx_cols = []
    lane_iota = jax.lax.broadcasted_iota(jnp.int32, (_T, _K), 1)
    for q in range(_NQ):
        cb = cb_ref[q]
        rc = jax.lax.dot_general(
            res.astype(jnp.bfloat16), cb.astype(jnp.bfloat16),
            (((1,), (1,)), ((), ())),
            preferred_element_type=jnp.float32)
        r2 = jnp.sum(res * res, axis=1, keepdims=True)
        c2 = jnp.sum(cb * cb, axis=1)[None, :]
        d2 = (r2 - 2.0 * rc) + c2
        m = jnp.min(d2, axis=1, keepdims=True)
        idx = jnp.min(jnp.where(d2 == m, lane_iota, _K),
                      axis=1, keepdims=True)
        onehot = (lane_iota == idx).astype(jnp.float32)
        quant = jax.lax.dot_general(
            onehot, cb, (((1,), (0,)), ((), ())),
            preferred_element_type=jnp.float32,
            precision=jax.lax.Precision.HIGHEST)
        diff = quant - res
        loss = loss + jnp.sum(diff * diff)
        qst = res + (quant - res)
        res = res - qst
        qsum = qsum + qst
        idx_cols.append(idx)

    z = jax.lax.dot_general(
        qsum.astype(jnp.bfloat16), wout_ref[...].astype(jnp.bfloat16),
        (((1,), (0,)), ((), ())),
        preferred_element_type=jnp.float32)
    z_ref[...] = z + bout_ref[...]
    idx_ref[...] = jnp.concatenate(idx_cols, axis=1)
    acc = loss_ref[...] + jnp.reshape(loss, (1, 1))
    scale = jnp.where(i == pl.num_programs(0) - 1,
                      jnp.float32(1.0 / (_N * _DV)), jnp.float32(1.0))
    loss_ref[...] = acc * scale


@functools.partial(jax.jit, static_argnames=("interpret",))
def kernel(x, W_in, b_in, codebooks, W_out, b_out, interpret=False):
    xf = x.reshape(_N, _D)
    grid = (_N // _T,)
    z, idx, loss = pl.pallas_call(
        _rvq_body,
        grid=grid,
        in_specs=[
            pl.BlockSpec((_T, _D), lambda i: (i, 0)),
            pl.BlockSpec((_D, _DV), lambda i: (0, 0)),
            pl.BlockSpec((1, _DV), lambda i: (0, 0)),
            pl.BlockSpec((_NQ, _K, _DV), lambda i: (0, 0, 0)),
            pl.BlockSpec((_DV, _D), lambda i: (0, 0)),
            pl.BlockSpec((1, _D), lambda i: (0, 0)),
        ],
        out_specs=[
            pl.BlockSpec((_T, _D), lambda i: (i, 0)),
            pl.BlockSpec((_T, _NQ), lambda i: (i, 0)),
            pl.BlockSpec((1, 1), lambda i: (0, 0)),
        ],
        out_shape=[
            jax.ShapeDtypeStruct((_N, _D), jnp.float32),
            jax.ShapeDtypeStruct((_N, _NQ), jnp.int32),
            jax.ShapeDtypeStruct((1, 1), jnp.float32),
        ],
        interpret=interpret,
    )(xf, W_in, b_in.reshape(1, _DV), codebooks, W_out, b_out.reshape(1, _D))
    return (z.reshape(_B, _L, _D), idx.reshape(_B, _L, _NQ), loss[0, 0])


# speedup vs baseline: 1.7775x; 1.4456x over previous
import functools

import jax
import jax.numpy as jnp
from jax.experimental import pallas as pl

_B, _L, _D = 32, 1024, 768
_DV, _K, _NQ = 64, 512, 4
_N = _B * _L
_T = 1024


def _rvq_body(x_ref, win_ref, bin_ref, cb_ref, wout_ref, bout_ref,
              z_ref, idx_ref, loss_ref):
    i = pl.program_id(0)

    @pl.when(i == 0)
    def _init():
        loss_ref[...] = jnp.zeros_like(loss_ref)

    xb = x_ref[...]
    xp = jax.lax.dot_general(
        xb.astype(jnp.bfloat16), win_ref[...].astype(jnp.bfloat16),
        (((1,), (0,)), ((), ())),
        preferred_element_type=jnp.float32)
    res = xp + bin_ref[...]

    qsum = jnp.zeros_like(res)
    loss = jnp.float32(0.0)
    idx_cols = []
    kiota = jax.lax.broadcasted_iota(jnp.int32, (_K, 1), 0)
    aug2 = jnp.concatenate(
        [jnp.ones((_K, 1), jnp.float32),
         (kiota & 255).astype(jnp.float32),
         (kiota & 256).astype(jnp.float32)], axis=1).astype(jnp.bfloat16)

    def _mm_bf16(a, b):
        return jax.lax.dot_general(
            a, b, (((1,), (0,)), ((), ())),
            preferred_element_type=jnp.float32)

    for q in range(_NQ):
        cb = cb_ref[q]
        cb_p0 = cb.astype(jnp.bfloat16)
        cb_r = cb - cb_p0.astype(jnp.float32)
        cb_p1 = cb_r.astype(jnp.bfloat16)
        cb_p2 = (cb_r - cb_p1.astype(jnp.float32)).astype(jnp.bfloat16)
        rc = jax.lax.dot_general(
            res.astype(jnp.bfloat16), cb_p0,
            (((1,), (1,)), ((), ())),
            preferred_element_type=jnp.float32)
        r2 = jnp.sum(res * res, axis=1, keepdims=True)
        c2 = jnp.sum(cb * cb, axis=1)[None, :]
        d2 = (r2 - 2.0 * rc) + c2
        m = jnp.min(d2, axis=1, keepdims=True)
        eqb = (d2 == m).astype(jnp.bfloat16)
        agg = _mm_bf16(eqb, aug2)
        cnt = agg[:, 0:1]
        quant = ((_mm_bf16(eqb, cb_p0) + _mm_bf16(eqb, cb_p1))
                 + _mm_bf16(eqb, cb_p2))

        def _slow(d2=d2, m=m, cb_p0=cb_p0, cb_p1=cb_p1, cb_p2=cb_p2):
            lane_iota = jax.lax.broadcasted_iota(
                jnp.int32, (_T, _K), 1).astype(jnp.float32)
            idxf = jnp.min(jnp.where(d2 == m, lane_iota, jnp.float32(_K)),
                           axis=1, keepdims=True)
            onehot = (lane_iota == idxf).astype(jnp.bfloat16)
            quant = ((_mm_bf16(onehot, cb_p0) + _mm_bf16(onehot, cb_p1))
                     + _mm_bf16(onehot, cb_p2))
            return idxf, quant

        def _fast(agg=agg, quant=quant):
            return agg[:, 1:2] + agg[:, 2:3], quant

        idxf, quant = jax.lax.cond(
            jnp.max(cnt) > 1.5, _slow, _fast)
        diff = quant - res
        loss = loss + jnp.sum(diff * diff)
        qst = res + (quant - res)
        res = res - qst
        qsum = qsum + qst
        idx_cols.append(idxf.astype(jnp.int32))

    z = jax.lax.dot_general(
        qsum.astype(jnp.bfloat16), wout_ref[...].astype(jnp.bfloat16),
        (((1,), (0,)), ((), ())),
        preferred_element_type=jnp.float32)
    z_ref[...] = z + bout_ref[...]
    idx_ref[...] = jnp.concatenate(idx_cols, axis=1)
    acc = loss_ref[...] + jnp.reshape(loss, (1, 1))
    scale = jnp.where(i == pl.num_programs(0) - 1,
                      jnp.float32(1.0 / (_N * _DV)), jnp.float32(1.0))
    loss_ref[...] = acc * scale


@functools.partial(jax.jit, static_argnames=("interpret",))
def kernel(x, W_in, b_in, codebooks, W_out, b_out, interpret=False):
    xf = x.reshape(_N, _D)
    grid = (_N // _T,)
    z, idx, loss = pl.pallas_call(
        _rvq_body,
        grid=grid,
        in_specs=[
            pl.BlockSpec((_T, _D), lambda i: (i, 0)),
            pl.BlockSpec((_D, _DV), lambda i: (0, 0)),
            pl.BlockSpec((1, _DV), lambda i: (0, 0)),
            pl.BlockSpec((_NQ, _K, _DV), lambda i: (0, 0, 0)),
            pl.BlockSpec((_DV, _D), lambda i: (0, 0)),
            pl.BlockSpec((1, _D), lambda i: (0, 0)),
        ],
        out_specs=[
            pl.BlockSpec((_T, _D), lambda i: (i, 0)),
            pl.BlockSpec((_T, _NQ), lambda i: (i, 0)),
            pl.BlockSpec((1, 1), lambda i: (0, 0)),
        ],
        out_shape=[
            jax.ShapeDtypeStruct((_N, _D), jnp.float32),
            jax.ShapeDtypeStruct((_N, _NQ), jnp.int32),
            jax.ShapeDtypeStruct((1, 1), jnp.float32),
        ],
        interpret=interpret,
    )(xf, W_in, b_in.reshape(1, _DV), codebooks, W_out, b_out.reshape(1, _D))
    return (z.reshape(_B, _L, _D), idx.reshape(_B, _L, _NQ), loss[0, 0])


# speedup vs baseline: 2.4651x; 1.3868x over previous
import functools

import jax
import jax.numpy as jnp
from jax.experimental import pallas as pl

_B, _L, _D = 32, 1024, 768
_DV, _K, _NQ = 64, 512, 4
_N = _B * _L
_T = 1024


def _rvq_body(x_ref, win_ref, bin_ref, cb_ref, wout_ref, bout_ref,
              z_ref, idx_ref, loss_ref):
    i = pl.program_id(0)

    @pl.when(i == 0)
    def _init():
        loss_ref[...] = jnp.zeros_like(loss_ref)

    xb = x_ref[...]
    xp = jax.lax.dot_general(
        xb.astype(jnp.bfloat16), win_ref[...].astype(jnp.bfloat16),
        (((1,), (0,)), ((), ())),
        preferred_element_type=jnp.float32)
    res = xp + bin_ref[...]

    qsum = jnp.zeros_like(res)
    loss = jnp.float32(0.0)
    idx_cols = []
    kiota = jax.lax.broadcasted_iota(jnp.int32, (_K, 1), 0)
    aug2 = jnp.concatenate(
        [jnp.ones((_K, 1), jnp.float32),
         (kiota & 255).astype(jnp.float32),
         (kiota & 256).astype(jnp.float32)], axis=1).astype(jnp.bfloat16)

    def _mm_bf16(a, b):
        return jax.lax.dot_general(
            a, b, (((1,), (0,)), ((), ())),
            preferred_element_type=jnp.float32)

    for q in range(_NQ):
        cb = cb_ref[q]
        cb_p0 = cb.astype(jnp.bfloat16)
        cb_r = cb - cb_p0.astype(jnp.float32)
        cb_p1 = cb_r.astype(jnp.bfloat16)
        cb_p2 = (cb_r - cb_p1.astype(jnp.float32)).astype(jnp.bfloat16)
        rc = jax.lax.dot_general(
            res.astype(jnp.bfloat16), cb_p0,
            (((1,), (1,)), ((), ())),
            preferred_element_type=jnp.float32)
        r2 = jnp.sum(res * res, axis=1, keepdims=True)
        c2 = jnp.sum(cb * cb, axis=1)[None, :]
        d2 = (r2 - 2.0 * rc) + c2
        m = jnp.min(d2, axis=1, keepdims=True)
        eqb = (d2 == m).astype(jnp.bfloat16)
        rhs = jnp.concatenate([cb_p0, cb_p1, cb_p2, aug2], axis=1)
        agg = _mm_bf16(eqb, rhs)
        cnt = agg[:, 3 * _DV:3 * _DV + 1]
        quant = ((agg[:, 0:_DV] + agg[:, _DV:2 * _DV])
                 + agg[:, 2 * _DV:3 * _DV])

        def _slow(d2=d2, m=m, cb_p0=cb_p0, cb_p1=cb_p1, cb_p2=cb_p2):
            lane_iota = jax.lax.broadcasted_iota(
                jnp.int32, (_T, _K), 1).astype(jnp.float32)
            idxf = jnp.min(jnp.where(d2 == m, lane_iota, jnp.float32(_K)),
                           axis=1, keepdims=True)
            onehot = (lane_iota == idxf).astype(jnp.bfloat16)
            quant = ((_mm_bf16(onehot, cb_p0) + _mm_bf16(onehot, cb_p1))
                     + _mm_bf16(onehot, cb_p2))
            return idxf, quant

        def _fast(agg=agg, quant=quant):
            return (agg[:, 3 * _DV + 1:3 * _DV + 2]
                    + agg[:, 3 * _DV + 2:3 * _DV + 3]), quant

        idxf, quant = jax.lax.cond(
            jnp.max(cnt) > 1.5, _slow, _fast)
        diff = quant - res
        loss = loss + jnp.sum(diff * diff)
        qst = res + (quant - res)
        res = res - qst
        qsum = qsum + qst
        idx_cols.append(idxf.astype(jnp.int32))

    z = jax.lax.dot_general(
        qsum.astype(jnp.bfloat16), wout_ref[...].astype(jnp.bfloat16),
        (((1,), (0,)), ((), ())),
        preferred_element_type=jnp.float32)
    z_ref[...] = z + bout_ref[...]
    idx_ref[...] = jnp.concatenate(idx_cols, axis=1)
    acc = loss_ref[...] + jnp.reshape(loss, (1, 1))
    scale = jnp.where(i == pl.num_programs(0) - 1,
                      jnp.float32(1.0 / (_N * _DV)), jnp.float32(1.0))
    loss_ref[...] = acc * scale


@functools.partial(jax.jit, static_argnames=("interpret",))
def kernel(x, W_in, b_in, codebooks, W_out, b_out, interpret=False):
    xf = x.reshape(_N, _D)
    grid = (_N // _T,)
    z, idx, loss = pl.pallas_call(
        _rvq_body,
        grid=grid,
        in_specs=[
            pl.BlockSpec((_T, _D), lambda i: (i, 0)),
            pl.BlockSpec((_D, _DV), lambda i: (0, 0)),
            pl.BlockSpec((1, _DV), lambda i: (0, 0)),
            pl.BlockSpec((_NQ, _K, _DV), lambda i: (0, 0, 0)),
            pl.BlockSpec((_DV, _D), lambda i: (0, 0)),
            pl.BlockSpec((1, _D), lambda i: (0, 0)),
        ],
        out_specs=[
            pl.BlockSpec((_T, _D), lambda i: (i, 0)),
            pl.BlockSpec((_T, _NQ), lambda i: (i, 0)),
            pl.BlockSpec((1, 1), lambda i: (0, 0)),
        ],
        out_shape=[
            jax.ShapeDtypeStruct((_N, _D), jnp.float32),
            jax.ShapeDtypeStruct((_N, _NQ), jnp.int32),
            jax.ShapeDtypeStruct((1, 1), jnp.float32),
        ],
        interpret=interpret,
    )(xf, W_in, b_in.reshape(1, _DV), codebooks, W_out, b_out.reshape(1, _D))
    return (z.reshape(_B, _L, _D), idx.reshape(_B, _L, _NQ), loss[0, 0])
